# baseline (speedup 1.0000x reference)
import functools

import jax
import jax.numpy as jnp
from jax import lax
from jax.experimental import pallas as pl
from jax.experimental.pallas import tpu as pltpu
from jax.experimental.pallas import tpu_sc as plsc

_NC = 2
_NS = 16
_NW = _NC * _NS

_DIM = 128
_N_USERS = 10000
_N_NEWS = 5000
_N_ENTITY = 10000

_G_CH = 128
_G_NBUF = 3


def _gather_body(n_groups, per_tile, tab, idxh, out, idx_v, rows_v,
                 gsem, ssem):
    c = lax.axis_index("c")
    s = lax.axis_index("s")
    wid = s * _NC + c
    base = wid * per_tile

    pltpu.sync_copy(idxh.at[pl.ds(base, per_tile)], idx_v)

    def fire(g, bset):
        cbase = g * (_G_NBUF * _G_CH)
        for b in range(_G_NBUF):
            pltpu.async_copy(
                tab.at[idx_v.at[pl.ds(cbase + b * _G_CH, _G_CH)]],
                rows_v.at[bset, b], gsem)

    def drain_and_store(g, bset):
        cbase = g * (_G_NBUF * _G_CH)
        for b in range(_G_NBUF):
            pltpu.make_async_copy(
                tab.at[idx_v.at[pl.ds(cbase + b * _G_CH, _G_CH)]],
                rows_v.at[bset, b], gsem).wait()
        for b in range(_G_NBUF):
            pltpu.async_copy(
                rows_v.at[bset, b],
                out.at[pl.ds(base + cbase + b * _G_CH, _G_CH)], ssem)

    def wait_stores(g, bset):
        cbase = g * (_G_NBUF * _G_CH)
        for b in range(_G_NBUF):
            pltpu.make_async_copy(
                rows_v.at[bset, b],
                out.at[pl.ds(base + cbase + b * _G_CH, _G_CH)], ssem).wait()

    fire(0, 0)

    def grp(g, carry):
        bset = lax.rem(g, 2)

        @pl.when(g >= 1)
        def _w():
            wait_stores(g - 1, 1 - bset)

        @pl.when(g + 1 < n_groups)
        def _f():
            fire(g + 1, 1 - bset)

        drain_and_store(g, bset)
        return carry

    lax.fori_loop(0, n_groups, grp, 0)
    wait_stores(n_groups - 1, lax.rem(n_groups - 1, 2))


def _sc_gather(table, idx):
    tot = idx.shape[0]
    per_tile = tot // _NW
    n_groups = per_tile // (_G_NBUF * _G_CH)
    mesh = plsc.VectorSubcoreMesh(core_axis_name="c", subcore_axis_name="s")
    f = pl.kernel(
        functools.partial(_gather_body, n_groups, per_tile),
        mesh=mesh,
        out_type=jax.ShapeDtypeStruct((tot, _DIM), jnp.float32),
        scratch_types=[
            pltpu.VMEM((per_tile,), jnp.int32),
            pltpu.VMEM((2, _G_NBUF, _G_CH, _DIM), jnp.float32),
            pltpu.SemaphoreType.DMA,
            pltpu.SemaphoreType.DMA,
        ],
    )
    return f(table, idx)


_E_TOT = 102400
_E_PT = _E_TOT // _NW
_E_CH = 128
_E_NCH = _E_PT // _E_CH
_E_NZT = 10
_E_RZ = _N_USERS // _E_NZT


def _edge_body(a_hbm, cols_hbm, rows3_hbm, vals_hbm, zeros_hbm, parts,
               cols_v, vals_v, rowi_v, buf, acc, sem):
    c = lax.axis_index("c")
    s = lax.axis_index("s")
    wid = s * _NC + c
    base = wid * _E_PT
    pltpu.sync_copy(cols_hbm.at[pl.ds(base, _E_PT)], cols_v)
    pltpu.sync_copy(vals_hbm.at[pl.ds(base, _E_PT)], vals_v)
    pltpu.sync_copy(rows3_hbm.at[wid], rowi_v)
    @pl.when(s < _E_NZT)
    def _zero():
        pltpu.sync_copy(zeros_hbm.at[pl.ds(s * _E_RZ, _E_RZ)],
                        acc.at[pl.ds(s * _E_RZ, _E_RZ)])

    plsc.subcore_barrier()

    def chunk(ci, carry):
        off = ci * _E_CH
        pltpu.async_copy(a_hbm.at[cols_v.at[pl.ds(off, _E_CH)]], buf,
                         sem).wait()

        def qb(q, cc):
            v16 = vals_v[pl.ds(off + q * 16, 16)]
            for t in range(16):
                vs = v16[t]
                r = q * 16 + t
                for g in range(8):
                    sl = pl.ds(g * 16, 16)
                    buf[r, sl] = buf[r, sl] * vs
            return cc

        lax.fori_loop(0, _E_CH // 16, qb, 0)
        pltpu.sync_copy(buf, acc.at[rowi_v.at[ci]], add=True)
        return carry

    lax.fori_loop(0, _E_NCH, chunk, 0)
    plsc.subcore_barrier()

    @pl.when(s < _E_NZT)
    def _copy_out():
        pltpu.sync_copy(acc.at[pl.ds(s * _E_RZ, _E_RZ)],
                        parts.at[c, pl.ds(s * _E_RZ, _E_RZ)])


def _sc_edges(news_agg, cols_p, rows3, vals_p, zeros):
    mesh = plsc.VectorSubcoreMesh(core_axis_name="c", subcore_axis_name="s")
    f = pl.kernel(
        _edge_body,
        mesh=mesh,
        out_type=jax.ShapeDtypeStruct((_NC, _N_USERS, _DIM), jnp.float32),
        scratch_types=[
            pltpu.VMEM((_E_PT,), jnp.int32),
            pltpu.VMEM((_E_PT,), jnp.float32),
            pltpu.VMEM((_E_NCH, _E_CH), jnp.int32),
            pltpu.VMEM((_E_CH, _DIM), jnp.float32),
            pltpu.VMEM_SHARED((_N_USERS, _DIM), jnp.float32),
            pltpu.SemaphoreType.DMA,
        ],
    )
    return f(news_agg, cols_p, rows3, vals_p, zeros)


def _attn_body(h_ref, t_ref, rel_ref, r_ref, agg_ref, res_ref):
    H = h_ref[...]
    T = t_ref[...]
    RI = rel_ref[...]
    R = r_ref[...]
    B = H.shape[0]
    K = T.shape[1]
    RI3 = RI[:, :, None]
    RR = jnp.zeros((B, K, _DIM), jnp.float32)
    for j in range(10):
        sel = (RI3 == j).astype(jnp.float32)
        RR = RR + sel * R[j, :][None, None, :]
    G = jnp.abs(H) * jnp.sqrt(jnp.sum(RR * RR, axis=1))
    TRb = (T * RR).astype(jnp.bfloat16).astype(jnp.float32)
    Gb = G.astype(jnp.bfloat16).astype(jnp.float32)
    att = jnp.sum(TRb * Gb[:, None, :], axis=2)
    att = att * att
    m = jnp.max(att, axis=1, keepdims=True)
    p = jnp.exp(att - m)
    w = p / jnp.sum(p, axis=1, keepdims=True)
    agg = jnp.sum(w[..., None] * T, axis=1)
    agg_ref[...] = agg
    nrm = jnp.sqrt(jnp.sum(agg * agg, axis=1, keepdims=True))
    res_ref[...] = H + 2.0 * (agg / jnp.maximum(nrm, 1e-12))


def _attn(heads, tails, relidx, R, block):
    M = heads.shape[0]
    K = tails.shape[1]
    grid = (M // block,)
    return pl.pallas_call(
        _attn_body,
        grid=grid,
        in_specs=[
            pl.BlockSpec((block, _DIM), lambda i: (i, 0)),
            pl.BlockSpec((block, K, _DIM), lambda i: (i, 0, 0)),
            pl.BlockSpec((block, K), lambda i: (i, 0)),
            pl.BlockSpec((10, _DIM), lambda i: (0, 0)),
        ],
        out_specs=[
            pl.BlockSpec((block, _DIM), lambda i: (i, 0)),
            pl.BlockSpec((block, _DIM), lambda i: (i, 0)),
        ],
        out_shape=[
            jax.ShapeDtypeStruct((M, _DIM), jnp.float32),
            jax.ShapeDtypeStruct((M, _DIM), jnp.float32),
        ],
    )(heads, tails, relidx, R)


def _user_body(u_ref, a_ref, p0_ref, p1_ref, out_ref):
    U = u_ref[...]
    A = a_ref[...]
    logits = lax.dot_general(U, A, (((1,), (1,)), ((), ())),
                             preferred_element_type=jnp.float32)
    m = jnp.max(logits, axis=1, keepdims=True)
    p = jnp.exp(logits - m)
    ssum = jnp.sum(p, axis=1, keepdims=True)
    score = p / ssum
    num = lax.dot_general(score, A, (((1,), (0,)), ((), ())),
                          preferred_element_type=jnp.float32)
    ua = p0_ref[...] + p1_ref[...]
    uaf = ua + num * ua
    nrm = jnp.sqrt(jnp.sum(uaf * uaf, axis=1, keepdims=True))
    out_ref[...] = U + 2.0 * (uaf / jnp.maximum(nrm, 1e-12))


def _user_update(U, A, p0, p1, block=200):
    M = U.shape[0]
    grid = (M // block,)
    return pl.pallas_call(
        _user_body,
        grid=grid,
        in_specs=[
            pl.BlockSpec((block, _DIM), lambda i: (i, 0)),
            pl.BlockSpec((_N_NEWS, _DIM), lambda i: (0, 0)),
            pl.BlockSpec((block, _DIM), lambda i: (i, 0)),
            pl.BlockSpec((block, _DIM), lambda i: (i, 0)),
        ],
        out_specs=pl.BlockSpec((block, _DIM), lambda i: (i, 0)),
        out_shape=jax.ShapeDtypeStruct((M, _DIM), jnp.float32),
    )(U, A, p0, p1)


_TOPK = 10


def _topk_body(xb_ref, xf_ref, kv_ref, ki_ref):
    Xf = xf_ref[...]
    nf = jnp.sqrt(jnp.sum(Xf * Xf, axis=1, keepdims=True))
    Xfn = Xf / nf
    Xb = xb_ref[...]
    nb = jnp.sqrt(jnp.sum(Xb * Xb, axis=1, keepdims=True))
    Xbn = Xb / nb
    sim = lax.dot_general(Xbn, Xfn, (((1,), (1,)), ((), ())),
                          preferred_element_type=jnp.float32)
    B = sim.shape[0]
    N = sim.shape[1]
    cols = lax.broadcasted_iota(jnp.int32, (B, N), 1)
    vals = []
    inds = []
    cur = sim
    for _ in range(_TOPK):
        m = jnp.max(cur, axis=1, keepdims=True)
        idx = jnp.min(jnp.where(cur == m, cols, jnp.int32(1 << 30)),
                      axis=1, keepdims=True)
        vals.append(m)
        inds.append(idx)
        cur = jnp.where(cols == idx, jnp.float32(-3e38), cur)
    kv_ref[...] = jnp.concatenate(vals, axis=1)
    ki_ref[...] = jnp.concatenate(inds, axis=1)


def _topk(X, block=200):
    M = X.shape[0]
    grid = (M // block,)
    return pl.pallas_call(
        _topk_body,
        grid=grid,
        in_specs=[
            pl.BlockSpec((block, _DIM), lambda i: (i, 0)),
            pl.BlockSpec((M, _DIM), lambda i: (0, 0)),
        ],
        out_specs=[
            pl.BlockSpec((block, _TOPK), lambda i: (i, 0)),
            pl.BlockSpec((block, _TOPK), lambda i: (i, 0)),
        ],
        out_shape=[
            jax.ShapeDtypeStruct((M, _TOPK), jnp.float32),
            jax.ShapeDtypeStruct((M, _TOPK), jnp.int32),
        ],
    )(X, X)


def _adj_body(kv1b_ref, ki1b_ref, kv1t_ref, kv2b_ref, ki2b_ref, kv2t_ref,
              out_ref):
    B = kv1b_ref.shape[0]
    N = out_ref.shape[1]

    def dcol(kv):
        rs = jnp.sum(kv, axis=1, keepdims=True)
        safe = jnp.where(rs > 0, rs, 1.0)
        return jnp.where(rs > 0, 1.0 / jnp.sqrt(safe), 0.0)

    def drow(kvt):
        rs = jnp.sum(kvt, axis=0, keepdims=True)
        safe = jnp.where(rs > 0, rs, 1.0)
        return jnp.where(rs > 0, 1.0 / jnp.sqrt(safe), 0.0)

    cols = lax.broadcasted_iota(jnp.int32, (B, N), 1)

    def half(kvb, kib, kvt):
        S = jnp.zeros((B, N), jnp.float32)
        for j in range(_TOPK):
            S = S + jnp.where(cols == kib[:, j:j + 1],
                              kvb[:, j:j + 1], 0.0)
        return dcol(kvb) * S * drow(kvt)

    out_ref[...] = 0.5 * half(kv1b_ref[...], ki1b_ref[...], kv1t_ref[...]) \
        + 0.5 * half(kv2b_ref[...], ki2b_ref[...], kv2t_ref[...])


def _assemble_adj(kv1, ki1, kv2, ki2, block=200):
    M = kv1.shape[0]
    grid = (M // block,)
    return pl.pallas_call(
        _adj_body,
        grid=grid,
        in_specs=[
            pl.BlockSpec((block, _TOPK), lambda i: (i, 0)),
            pl.BlockSpec((block, _TOPK), lambda i: (i, 0)),
            pl.BlockSpec((_TOPK, M), lambda i: (0, 0)),
            pl.BlockSpec((block, _TOPK), lambda i: (i, 0)),
            pl.BlockSpec((block, _TOPK), lambda i: (i, 0)),
            pl.BlockSpec((_TOPK, M), lambda i: (0, 0)),
        ],
        out_specs=pl.BlockSpec((block, M), lambda i: (i, 0)),
        out_shape=jax.ShapeDtypeStruct((M, M), jnp.float32),
    )(kv1, ki1, kv1.T, kv2, ki2, kv2.T)


def kernel(user_embedding, news_embedding, entity_embedding,
           relation_embedding, interact_indices, interact_values,
           news_entities, news_relations, neigh_entities, neigh_relations):
    U = user_embedding
    N = news_embedding
    E = entity_embedding
    R = relation_embedding

    unit = _NW * _G_NBUF * _G_CH

    def pad_idx(flat):
        n = flat.shape[0]
        tot = ((n + unit - 1) // unit) * unit
        return jnp.concatenate([flat, jnp.zeros((tot - n,), jnp.int32)]), n

    idx_news, n_news_idx = pad_idx(news_entities.reshape(-1))
    idx_ent, n_ent_idx = pad_idx(neigh_entities.reshape(-1))
    T_news = _sc_gather(E, idx_news)[:n_news_idx].reshape(
        _N_NEWS, 20, _DIM)
    T_ent = _sc_gather(E, idx_ent)[:n_ent_idx].reshape(
        _N_ENTITY, 20, _DIM)

    news_agg, news_res = _attn(N, T_news, news_relations, R, block=200)
    _, ent_res = _attn(E, T_ent, neigh_relations, R, block=200)

    rows = interact_indices[0]
    cols = interact_indices[1]
    nnz = rows.shape[0]
    padn = _E_TOT - nnz
    rows3 = jnp.concatenate([rows, jnp.zeros((padn,), jnp.int32)]).reshape(
        _NW, _E_NCH, _E_CH)
    cols_p = jnp.concatenate([cols, jnp.zeros((padn,), jnp.int32)])
    vals_p = jnp.concatenate([interact_values,
                              jnp.zeros((padn,), jnp.float32)])
    zeros = jnp.zeros((_N_USERS, _DIM), jnp.float32)
    parts = _sc_edges(news_agg, cols_p, rows3, vals_p, zeros)

    user_res = _user_update(U, news_agg, parts[0], parts[1])

    kv1, ki1 = _topk(news_res)
    kv2, ki2 = _topk(N)
    item_adj = _assemble_adj(kv1, ki1, kv2, ki2)

    return ent_res, user_res, news_res, item_adj


# speedup vs baseline: 1.0708x; 1.0708x over previous
import functools

import jax
import jax.numpy as jnp
from jax import lax
from jax.experimental import pallas as pl
from jax.experimental.pallas import tpu as pltpu
from jax.experimental.pallas import tpu_sc as plsc

_NC = 2
_NS = 16
_NW = _NC * _NS

_DIM = 128
_N_USERS = 10000
_N_NEWS = 5000
_N_ENTITY = 10000

_G_CH = 128
_G_NBUF = 3


def _gather_body(n_groups, per_tile, tab, idxh, out, idx_v, rows_v,
                 gsem, ssem):
    c = lax.axis_index("c")
    s = lax.axis_index("s")
    wid = s * _NC + c
    base = wid * per_tile

    pltpu.sync_copy(idxh.at[pl.ds(base, per_tile)], idx_v)

    def fire(g, bset):
        cbase = g * (_G_NBUF * _G_CH)
        for b in range(_G_NBUF):
            pltpu.async_copy(
                tab.at[idx_v.at[pl.ds(cbase + b * _G_CH, _G_CH)]],
                rows_v.at[bset, b], gsem)

    def drain_and_store(g, bset):
        cbase = g * (_G_NBUF * _G_CH)
        for b in range(_G_NBUF):
            pltpu.make_async_copy(
                tab.at[idx_v.at[pl.ds(cbase + b * _G_CH, _G_CH)]],
                rows_v.at[bset, b], gsem).wait()
        for b in range(_G_NBUF):
            pltpu.async_copy(
                rows_v.at[bset, b],
                out.at[pl.ds(base + cbase + b * _G_CH, _G_CH)], ssem)

    def wait_stores(g, bset):
        cbase = g * (_G_NBUF * _G_CH)
        for b in range(_G_NBUF):
            pltpu.make_async_copy(
                rows_v.at[bset, b],
                out.at[pl.ds(base + cbase + b * _G_CH, _G_CH)], ssem).wait()

    fire(0, 0)

    def grp(g, carry):
        bset = lax.rem(g, 2)

        @pl.when(g >= 1)
        def _w():
            wait_stores(g - 1, 1 - bset)

        @pl.when(g + 1 < n_groups)
        def _f():
            fire(g + 1, 1 - bset)

        drain_and_store(g, bset)
        return carry

    lax.fori_loop(0, n_groups, grp, 0)
    wait_stores(n_groups - 1, lax.rem(n_groups - 1, 2))


def _sc_gather(table, idx):
    tot = idx.shape[0]
    per_tile = tot // _NW
    n_groups = per_tile // (_G_NBUF * _G_CH)
    mesh = plsc.VectorSubcoreMesh(core_axis_name="c", subcore_axis_name="s")
    f = pl.kernel(
        functools.partial(_gather_body, n_groups, per_tile),
        mesh=mesh,
        out_type=jax.ShapeDtypeStruct((tot, _DIM), jnp.float32),
        scratch_types=[
            pltpu.VMEM((per_tile,), jnp.int32),
            pltpu.VMEM((2, _G_NBUF, _G_CH, _DIM), jnp.float32),
            pltpu.SemaphoreType.DMA,
            pltpu.SemaphoreType.DMA,
        ],
    )
    return f(table, idx)


_E_TOT = 102400
_E_PT = _E_TOT // _NW
_E_CH = 128
_E_NCH = _E_PT // _E_CH
_E_NZT = 10
_E_RZ = _N_USERS // _E_NZT


def _edge_body(a_hbm, cols_hbm, rows3_hbm, vals_hbm, zeros_hbm, parts,
               cols_v, vals_v, rowi_v, buf, acc, sem):
    c = lax.axis_index("c")
    s = lax.axis_index("s")
    wid = s * _NC + c
    base = wid * _E_PT
    pltpu.sync_copy(cols_hbm.at[pl.ds(base, _E_PT)], cols_v)
    pltpu.sync_copy(vals_hbm.at[pl.ds(base, _E_PT)], vals_v)
    pltpu.sync_copy(rows3_hbm.at[wid], rowi_v)
    @pl.when(s < _E_NZT)
    def _zero():
        pltpu.sync_copy(zeros_hbm.at[pl.ds(s * _E_RZ, _E_RZ)],
                        acc.at[pl.ds(s * _E_RZ, _E_RZ)])

    plsc.subcore_barrier()

    def chunk(ci, carry):
        off = ci * _E_CH
        pltpu.async_copy(a_hbm.at[cols_v.at[pl.ds(off, _E_CH)]], buf,
                         sem).wait()

        def qb(q, cc):
            v16 = vals_v[pl.ds(off + q * 16, 16)]
            for t in range(16):
                vs = v16[t]
                r = q * 16 + t
                for g in range(8):
                    sl = pl.ds(g * 16, 16)
                    buf[r, sl] = buf[r, sl] * vs
            return cc

        lax.fori_loop(0, _E_CH // 16, qb, 0)
        pltpu.sync_copy(buf, acc.at[rowi_v.at[ci]], add=True)
        return carry

    lax.fori_loop(0, _E_NCH, chunk, 0)
    plsc.subcore_barrier()

    @pl.when(s < _E_NZT)
    def _copy_out():
        pltpu.sync_copy(acc.at[pl.ds(s * _E_RZ, _E_RZ)],
                        parts.at[c, pl.ds(s * _E_RZ, _E_RZ)])


def _sc_edges(news_agg, cols_p, rows3, vals_p, zeros):
    mesh = plsc.VectorSubcoreMesh(core_axis_name="c", subcore_axis_name="s")
    f = pl.kernel(
        _edge_body,
        mesh=mesh,
        out_type=jax.ShapeDtypeStruct((_NC, _N_USERS, _DIM), jnp.float32),
        scratch_types=[
            pltpu.VMEM((_E_PT,), jnp.int32),
            pltpu.VMEM((_E_PT,), jnp.float32),
            pltpu.VMEM((_E_NCH, _E_CH), jnp.int32),
            pltpu.VMEM((_E_CH, _DIM), jnp.float32),
            pltpu.VMEM_SHARED((_N_USERS, _DIM), jnp.float32),
            pltpu.SemaphoreType.DMA,
        ],
    )
    return f(news_agg, cols_p, rows3, vals_p, zeros)


def _attn_body(h_ref, t_ref, rel_ref, r_ref, agg_ref, res_ref):
    H = h_ref[...]
    T = t_ref[...]
    RI = rel_ref[...]
    R = r_ref[...]
    B = H.shape[0]
    K = T.shape[1]
    RI3 = RI[:, :, None]
    RR = jnp.zeros((B, K, _DIM), jnp.float32)
    for j in range(10):
        sel = (RI3 == j).astype(jnp.float32)
        RR = RR + sel * R[j, :][None, None, :]
    G = jnp.abs(H) * jnp.sqrt(jnp.sum(RR * RR, axis=1))
    TRb = (T * RR).astype(jnp.bfloat16).astype(jnp.float32)
    Gb = G.astype(jnp.bfloat16).astype(jnp.float32)
    att = jnp.sum(TRb * Gb[:, None, :], axis=2)
    att = att * att
    m = jnp.max(att, axis=1, keepdims=True)
    p = jnp.exp(att - m)
    w = p / jnp.sum(p, axis=1, keepdims=True)
    agg = jnp.sum(w[..., None] * T, axis=1)
    agg_ref[...] = agg
    nrm = jnp.sqrt(jnp.sum(agg * agg, axis=1, keepdims=True))
    res_ref[...] = H + 2.0 * (agg / jnp.maximum(nrm, 1e-12))


def _attn(heads, tails, relidx, R, block):
    M = heads.shape[0]
    K = tails.shape[1]
    grid = (M // block,)
    return pl.pallas_call(
        _attn_body,
        grid=grid,
        in_specs=[
            pl.BlockSpec((block, _DIM), lambda i: (i, 0)),
            pl.BlockSpec((block, K, _DIM), lambda i: (i, 0, 0)),
            pl.BlockSpec((block, K), lambda i: (i, 0)),
            pl.BlockSpec((10, _DIM), lambda i: (0, 0)),
        ],
        out_specs=[
            pl.BlockSpec((block, _DIM), lambda i: (i, 0)),
            pl.BlockSpec((block, _DIM), lambda i: (i, 0)),
        ],
        out_shape=[
            jax.ShapeDtypeStruct((M, _DIM), jnp.float32),
            jax.ShapeDtypeStruct((M, _DIM), jnp.float32),
        ],
    )(heads, tails, relidx, R)


def _user_body(u_ref, a_ref, p0_ref, p1_ref, out_ref):
    U = u_ref[...]
    A = a_ref[...]
    logits = lax.dot_general(U, A, (((1,), (1,)), ((), ())),
                             preferred_element_type=jnp.float32)
    m = jnp.max(logits, axis=1, keepdims=True)
    p = jnp.exp(logits - m)
    ssum = jnp.sum(p, axis=1, keepdims=True)
    score = p / ssum
    num = lax.dot_general(score, A, (((1,), (0,)), ((), ())),
                          preferred_element_type=jnp.float32)
    ua = p0_ref[...] + p1_ref[...]
    uaf = ua + num * ua
    nrm = jnp.sqrt(jnp.sum(uaf * uaf, axis=1, keepdims=True))
    out_ref[...] = U + 2.0 * (uaf / jnp.maximum(nrm, 1e-12))


def _user_update(U, A, p0, p1, block=200):
    M = U.shape[0]
    grid = (M // block,)
    return pl.pallas_call(
        _user_body,
        grid=grid,
        in_specs=[
            pl.BlockSpec((block, _DIM), lambda i: (i, 0)),
            pl.BlockSpec((_N_NEWS, _DIM), lambda i: (0, 0)),
            pl.BlockSpec((block, _DIM), lambda i: (i, 0)),
            pl.BlockSpec((block, _DIM), lambda i: (i, 0)),
        ],
        out_specs=pl.BlockSpec((block, _DIM), lambda i: (i, 0)),
        out_shape=jax.ShapeDtypeStruct((M, _DIM), jnp.float32),
    )(U, A, p0, p1)


_TOPK = 10


def _topk_body(xb_ref, xf_ref, kv_ref, ki_ref):
    Xf = xf_ref[...]
    nf = jnp.sqrt(jnp.sum(Xf * Xf, axis=1, keepdims=True))
    Xfn = Xf / nf
    Xb = xb_ref[...]
    nb = jnp.sqrt(jnp.sum(Xb * Xb, axis=1, keepdims=True))
    Xbn = Xb / nb
    sim = lax.dot_general(Xbn, Xfn, (((1,), (1,)), ((), ())),
                          preferred_element_type=jnp.float32)
    B = sim.shape[0]
    N = sim.shape[1]
    cols = lax.broadcasted_iota(jnp.int32, (B, N), 1)
    vals = []
    inds = []
    cur = sim
    for _ in range(_TOPK):
        m = jnp.max(cur, axis=1, keepdims=True)
        idx = jnp.min(jnp.where(cur == m, cols, jnp.int32(1 << 30)),
                      axis=1, keepdims=True)
        vals.append(m)
        inds.append(idx)
        cur = jnp.where(cols == idx, jnp.float32(-3e38), cur)
    kv_ref[...] = jnp.concatenate(vals, axis=1)
    ki_ref[...] = jnp.concatenate(inds, axis=1)


def _topk(X, block=200):
    M = X.shape[0]
    grid = (M // block,)
    return pl.pallas_call(
        _topk_body,
        grid=grid,
        in_specs=[
            pl.BlockSpec((block, _DIM), lambda i: (i, 0)),
            pl.BlockSpec((M, _DIM), lambda i: (0, 0)),
        ],
        out_specs=[
            pl.BlockSpec((block, _TOPK), lambda i: (i, 0)),
            pl.BlockSpec((block, _TOPK), lambda i: (i, 0)),
        ],
        out_shape=[
            jax.ShapeDtypeStruct((M, _TOPK), jnp.float32),
            jax.ShapeDtypeStruct((M, _TOPK), jnp.int32),
        ],
    )(X, X)


def _adj_body(kv1b_ref, ki1b_ref, kv1t_ref, kv2b_ref, ki2b_ref, kv2t_ref,
              out_ref):
    B = kv1b_ref.shape[0]
    N = out_ref.shape[1]

    def dcol(kv):
        rs = jnp.sum(kv, axis=1, keepdims=True)
        safe = jnp.where(rs > 0, rs, 1.0)
        return jnp.where(rs > 0, 1.0 / jnp.sqrt(safe), 0.0)

    def drow(kvt):
        rs = jnp.sum(kvt, axis=0, keepdims=True)
        safe = jnp.where(rs > 0, rs, 1.0)
        return jnp.where(rs > 0, 1.0 / jnp.sqrt(safe), 0.0)

    cols = lax.broadcasted_iota(jnp.int32, (B, N), 1)

    def half(kvb, kib, kvt):
        S = jnp.zeros((B, N), jnp.float32)
        for j in range(_TOPK):
            S = S + jnp.where(cols == kib[:, j:j + 1],
                              kvb[:, j:j + 1], 0.0)
        return dcol(kvb) * S * drow(kvt)

    out_ref[...] = 0.5 * half(kv1b_ref[...], ki1b_ref[...], kv1t_ref[...]) \
        + 0.5 * half(kv2b_ref[...], ki2b_ref[...], kv2t_ref[...])


def _assemble_adj(kv1, ki1, kv2, ki2, block=200):
    M = kv1.shape[0]
    grid = (M // block,)
    return pl.pallas_call(
        _adj_body,
        grid=grid,
        in_specs=[
            pl.BlockSpec((block, _TOPK), lambda i: (i, 0)),
            pl.BlockSpec((block, _TOPK), lambda i: (i, 0)),
            pl.BlockSpec((_TOPK, M), lambda i: (0, 0)),
            pl.BlockSpec((block, _TOPK), lambda i: (i, 0)),
            pl.BlockSpec((block, _TOPK), lambda i: (i, 0)),
            pl.BlockSpec((_TOPK, M), lambda i: (0, 0)),
        ],
        out_specs=pl.BlockSpec((block, M), lambda i: (i, 0)),
        out_shape=jax.ShapeDtypeStruct((M, M), jnp.float32),
    )(kv1, ki1, kv1.T, kv2, ki2, kv2.T)


def kernel(user_embedding, news_embedding, entity_embedding,
           relation_embedding, interact_indices, interact_values,
           news_entities, news_relations, neigh_entities, neigh_relations):
    U = user_embedding
    N = news_embedding
    E = entity_embedding
    R = relation_embedding

    unit = _NW * _G_NBUF * _G_CH

    def pad_idx(flat):
        n = flat.shape[0]
        tot = ((n + unit - 1) // unit) * unit
        return jnp.concatenate([flat, jnp.zeros((tot - n,), jnp.int32)]), n

    n_news_idx = _N_NEWS * 20
    n_ent_idx = _N_ENTITY * 20
    idx_all, _ = pad_idx(jnp.concatenate([news_entities.reshape(-1),
                                          neigh_entities.reshape(-1)]))
    T_all = _sc_gather(E, idx_all)
    T_news = T_all[:n_news_idx].reshape(_N_NEWS, 20, _DIM)
    T_ent = T_all[n_news_idx:n_news_idx + n_ent_idx].reshape(
        _N_ENTITY, 20, _DIM)

    news_agg, news_res = _attn(N, T_news, news_relations, R, block=200)
    _, ent_res = _attn(E, T_ent, neigh_relations, R, block=200)

    rows = interact_indices[0]
    cols = interact_indices[1]
    nnz = rows.shape[0]
    padn = _E_TOT - nnz
    rows3 = jnp.concatenate([rows, jnp.zeros((padn,), jnp.int32)]).reshape(
        _NW, _E_NCH, _E_CH)
    cols_p = jnp.concatenate([cols, jnp.zeros((padn,), jnp.int32)])
    vals_p = jnp.concatenate([interact_values,
                              jnp.zeros((padn,), jnp.float32)])
    zeros = jnp.zeros((_N_USERS, _DIM), jnp.float32)
    parts = _sc_edges(news_agg, cols_p, rows3, vals_p, zeros)

    user_res = _user_update(U, news_agg, parts[0], parts[1])

    kv1, ki1 = _topk(news_res)
    kv2, ki2 = _topk(N)
    item_adj = _assemble_adj(kv1, ki1, kv2, ki2)

    return ent_res, user_res, news_res, item_adj
